# baseline (device time: 86572 ns/iter reference)
import jax
import jax.numpy as jnp
from jax import lax
from jax.experimental import pallas as pl
from jax.experimental.pallas import tpu as pltpu

N_DEV = 4
N_HOPS = N_DEV - 1


def kernel(x, router_W, route_idx, expert_W, shared_W):
    n_tok, d = x.shape
    e_loc, _, h = expert_W.shape

    def body(x_ref, rw_ref, idx_ref, ew_ref, sw_ref, out_ref,
             comm_ref, send_sems, recv_sems):
        my = lax.axis_index("i")
        left = (my - 1) % N_DEV
        right = (my + 1) % N_DEV

        barrier_sem = pltpu.get_barrier_semaphore()
        for nbr in (left, right):
            pl.semaphore_signal(barrier_sem, inc=1, device_id=(nbr,),
                                device_id_type=pl.DeviceIdType.MESH)
        pl.semaphore_wait(barrier_sem, 2)

        xv = x_ref[:, :]

        scores = jnp.dot(xv, rw_ref[:, :], preferred_element_type=jnp.float32)
        m = jnp.max(scores, axis=-1, keepdims=True)
        p = jnp.exp(scores - m)
        p = p / jnp.sum(p, axis=-1, keepdims=True)
        idx = idx_ref[:, :]
        e_iota = lax.broadcasted_iota(jnp.int32, scores.shape, 1)
        gate = jnp.sum(jnp.where(e_iota == idx, p, 0.0), axis=-1, keepdims=True)

        xg = xv * gate
        parts = [jnp.where(idx == (my * e_loc + le), xg, 0.0)
                 for le in range(e_loc)]
        xs = jnp.concatenate(parts, axis=1)
        w = ew_ref[:, :, :].reshape(e_loc * d, h)
        partial = jnp.dot(xs, w, preferred_element_type=jnp.float32)

        shared = jnp.dot(xv, sw_ref[:, :], preferred_element_type=jnp.float32)
        comm_ref[0, :, :] = partial
        out_ref[:, :] = shared + partial

        for hop in range(N_HOPS):
            rdma = pltpu.make_async_remote_copy(
                src_ref=comm_ref.at[hop],
                dst_ref=comm_ref.at[hop + 1],
                send_sem=send_sems.at[hop],
                recv_sem=recv_sems.at[hop],
                device_id=(right,),
                device_id_type=pl.DeviceIdType.MESH,
            )
            rdma.start()
            rdma.wait()
            out_ref[:, :] += comm_ref[hop + 1, :, :]

    return pl.pallas_call(
        body,
        out_shape=jax.ShapeDtypeStruct((n_tok, h), jnp.float32),
        in_specs=[pl.BlockSpec(memory_space=pltpu.VMEM)] * 5,
        out_specs=pl.BlockSpec(memory_space=pltpu.VMEM),
        scratch_shapes=[
            pltpu.VMEM((N_DEV, n_tok, h), jnp.float32),
            pltpu.SemaphoreType.DMA((N_HOPS,)),
            pltpu.SemaphoreType.DMA((N_HOPS,)),
        ],
        compiler_params=pltpu.CompilerParams(collective_id=0),
    )(x, router_W, route_idx, expert_W, shared_W)


# device time: 39497 ns/iter; 2.1919x vs baseline; 2.1919x over previous
import jax
import jax.numpy as jnp
from jax import lax
from jax.experimental import pallas as pl
from jax.experimental.pallas import tpu as pltpu

N_DEV = 4


def kernel(x, router_W, route_idx, expert_W, shared_W):
    n_tok, d = x.shape
    e_loc, _, h = expert_W.shape
    ch = n_tok // N_DEV

    def body(x_ref, rw_ref, idx_ref, ew_ref, sw_ref, out_ref,
             p_ref, rs_buf, rs_send, rs_recv, ag_send, ag_recv):
        my = lax.axis_index("i")

        barrier_sem = pltpu.get_barrier_semaphore()
        for o in (1, 2, 3):
            pl.semaphore_signal(barrier_sem, inc=1,
                                device_id=((my + o) % N_DEV,),
                                device_id_type=pl.DeviceIdType.MESH)
        pl.semaphore_wait(barrier_sem, 3)

        xv = x_ref[:, :]

        scores = jnp.dot(xv, rw_ref[:, :], preferred_element_type=jnp.float32)
        m = jnp.max(scores, axis=-1, keepdims=True)
        p = jnp.exp(scores - m)
        p = p / jnp.sum(p, axis=-1, keepdims=True)
        idx = idx_ref[:, :]
        e_iota = lax.broadcasted_iota(jnp.int32, scores.shape, 1)
        gate = jnp.sum(jnp.where(e_iota == idx, p, 0.0), axis=-1, keepdims=True)

        xg = xv * gate
        parts = [jnp.where(idx == (my * e_loc + le), xg, 0.0)
                 for le in range(e_loc)]
        xs = jnp.concatenate(parts, axis=1)
        w = ew_ref[:, :, :].reshape(e_loc * d, h)
        p_ref[:, :] = jnp.dot(xs, w, preferred_element_type=jnp.float32)

        rs_rdmas = []
        for o in (1, 2, 3):
            peer = (my + o) % N_DEV
            rdma = pltpu.make_async_remote_copy(
                src_ref=p_ref.at[pl.ds(peer * ch, ch)],
                dst_ref=rs_buf.at[N_DEV - o],
                send_sem=rs_send.at[o - 1],
                recv_sem=rs_recv.at[N_DEV - o],
                device_id=(peer,),
                device_id_type=pl.DeviceIdType.MESH,
            )
            rdma.start()
            rs_rdmas.append(rdma)

        xseg = x_ref[pl.ds(my * ch, ch), :]
        acc = p_ref[pl.ds(my * ch, ch), :] + jnp.dot(
            xseg, sw_ref[:, :], preferred_element_type=jnp.float32)

        for k in (1, 2, 3):
            recv = pltpu.make_async_remote_copy(
                src_ref=p_ref.at[pl.ds(0, ch)],
                dst_ref=rs_buf.at[k],
                send_sem=rs_send.at[0],
                recv_sem=rs_recv.at[k],
                device_id=(my,),
                device_id_type=pl.DeviceIdType.MESH,
            )
            recv.wait_recv()
            acc = acc + rs_buf[k, :, :]

        out_ref[pl.ds(my * ch, ch), :] = acc

        ag_rdmas = []
        for o in (1, 2, 3):
            peer = (my + o) % N_DEV
            rdma = pltpu.make_async_remote_copy(
                src_ref=out_ref.at[pl.ds(my * ch, ch)],
                dst_ref=out_ref.at[pl.ds(my * ch, ch)],
                send_sem=ag_send.at[o - 1],
                recv_sem=ag_recv.at[N_DEV - o],
                device_id=(peer,),
                device_id_type=pl.DeviceIdType.MESH,
            )
            rdma.start()
            ag_rdmas.append(rdma)

        for k in (1, 2, 3):
            recv = pltpu.make_async_remote_copy(
                src_ref=out_ref.at[pl.ds(0, ch)],
                dst_ref=out_ref.at[pl.ds(((my + k) % N_DEV) * ch, ch)],
                send_sem=ag_send.at[0],
                recv_sem=ag_recv.at[k],
                device_id=(my,),
                device_id_type=pl.DeviceIdType.MESH,
            )
            recv.wait_recv()

        for rdma in rs_rdmas + ag_rdmas:
            rdma.wait_send()

    return pl.pallas_call(
        body,
        out_shape=jax.ShapeDtypeStruct((n_tok, h), jnp.float32),
        in_specs=[pl.BlockSpec(memory_space=pltpu.VMEM)] * 5,
        out_specs=pl.BlockSpec(memory_space=pltpu.VMEM),
        scratch_shapes=[
            pltpu.VMEM((n_tok, h), jnp.float32),
            pltpu.VMEM((N_DEV, ch, h), jnp.float32),
            pltpu.SemaphoreType.DMA((3,)),
            pltpu.SemaphoreType.DMA((N_DEV,)),
            pltpu.SemaphoreType.DMA((3,)),
            pltpu.SemaphoreType.DMA((N_DEV,)),
        ],
        compiler_params=pltpu.CompilerParams(collective_id=0),
    )(x, router_W, route_idx, expert_W, shared_W)


# device time: 38045 ns/iter; 2.2755x vs baseline; 1.0382x over previous
import jax
import jax.numpy as jnp
from jax import lax
from jax.experimental import pallas as pl
from jax.experimental.pallas import tpu as pltpu

N_DEV = 4


def kernel(x, router_W, route_idx, expert_W, shared_W):
    n_tok, d = x.shape
    e_loc, _, h = expert_W.shape
    ch = n_tok // N_DEV

    def body(x_ref, rw_ref, idx_ref, ew_ref, sw_ref, out_ref,
             xg_ref, send_buf, rs_buf, rs_send, rs_recv, ag_send, ag_recv):
        my = lax.axis_index("i")

        barrier_sem = pltpu.get_barrier_semaphore()
        for o in (1, 2, 3):
            pl.semaphore_signal(barrier_sem, inc=1,
                                device_id=((my + o) % N_DEV,),
                                device_id_type=pl.DeviceIdType.MESH)
        pl.semaphore_wait(barrier_sem, 3)

        xv = x_ref[:, :]

        scores = jnp.dot(xv, rw_ref[:, :], preferred_element_type=jnp.float32)
        m = jnp.max(scores, axis=-1, keepdims=True)
        p = jnp.exp(scores - m)
        p = p / jnp.sum(p, axis=-1, keepdims=True)
        idx = idx_ref[:, :]
        e_iota = lax.broadcasted_iota(jnp.int32, scores.shape, 1)
        gate = jnp.sum(jnp.where(e_iota == idx, p, 0.0), axis=-1, keepdims=True)
        xg_ref[:, :] = xv * gate

        w = ew_ref[:, :, :].reshape(e_loc * d, h)

        def chunk_partial(row0):
            xgc = xg_ref[pl.ds(row0, ch), :]
            idc = idx_ref[pl.ds(row0, ch), :]
            xsc = jnp.concatenate(
                [jnp.where(idc == (my * e_loc + le), xgc, 0.0)
                 for le in range(e_loc)], axis=1)
            return jnp.dot(xsc, w, preferred_element_type=jnp.float32)

        rs_rdmas = []
        for o in (1, 2, 3):
            peer = (my + o) % N_DEV
            send_buf[o - 1, :, :] = chunk_partial(peer * ch)
            rdma = pltpu.make_async_remote_copy(
                src_ref=send_buf.at[o - 1],
                dst_ref=rs_buf.at[N_DEV - o],
                send_sem=rs_send.at[o - 1],
                recv_sem=rs_recv.at[N_DEV - o],
                device_id=(peer,),
                device_id_type=pl.DeviceIdType.MESH,
            )
            rdma.start()
            rs_rdmas.append(rdma)

        xseg = x_ref[pl.ds(my * ch, ch), :]
        acc = chunk_partial(my * ch) + jnp.dot(
            xseg, sw_ref[:, :], preferred_element_type=jnp.float32)

        for k in (1, 2, 3):
            recv = pltpu.make_async_remote_copy(
                src_ref=send_buf.at[0],
                dst_ref=rs_buf.at[k],
                send_sem=rs_send.at[0],
                recv_sem=rs_recv.at[k],
                device_id=(my,),
                device_id_type=pl.DeviceIdType.MESH,
            )
            recv.wait_recv()
            acc = acc + rs_buf[k, :, :]

        out_ref[pl.ds(my * ch, ch), :] = acc

        ag_rdmas = []
        for o in (1, 2, 3):
            peer = (my + o) % N_DEV
            rdma = pltpu.make_async_remote_copy(
                src_ref=out_ref.at[pl.ds(my * ch, ch)],
                dst_ref=out_ref.at[pl.ds(my * ch, ch)],
                send_sem=ag_send.at[o - 1],
                recv_sem=ag_recv.at[N_DEV - o],
                device_id=(peer,),
                device_id_type=pl.DeviceIdType.MESH,
            )
            rdma.start()
            ag_rdmas.append(rdma)

        for k in (1, 2, 3):
            recv = pltpu.make_async_remote_copy(
                src_ref=out_ref.at[pl.ds(0, ch)],
                dst_ref=out_ref.at[pl.ds(((my + k) % N_DEV) * ch, ch)],
                send_sem=ag_send.at[0],
                recv_sem=ag_recv.at[k],
                device_id=(my,),
                device_id_type=pl.DeviceIdType.MESH,
            )
            recv.wait_recv()

        for rdma in rs_rdmas + ag_rdmas:
            rdma.wait_send()

    return pl.pallas_call(
        body,
        out_shape=jax.ShapeDtypeStruct((n_tok, h), jnp.float32),
        in_specs=[pl.BlockSpec(memory_space=pltpu.VMEM)] * 5,
        out_specs=pl.BlockSpec(memory_space=pltpu.VMEM),
        scratch_shapes=[
            pltpu.VMEM((n_tok, d), jnp.float32),
            pltpu.VMEM((3, ch, h), jnp.float32),
            pltpu.VMEM((N_DEV, ch, h), jnp.float32),
            pltpu.SemaphoreType.DMA((3,)),
            pltpu.SemaphoreType.DMA((N_DEV,)),
            pltpu.SemaphoreType.DMA((3,)),
            pltpu.SemaphoreType.DMA((N_DEV,)),
        ],
        compiler_params=pltpu.CompilerParams(collective_id=0),
    )(x, router_W, route_idx, expert_W, shared_W)


# device time: 27113 ns/iter; 3.1930x vs baseline; 1.4032x over previous
import jax
import jax.numpy as jnp
from jax import lax
from jax.experimental import pallas as pl
from jax.experimental.pallas import tpu as pltpu

N_DEV = 4


def kernel(x, router_W, route_idx, expert_W, shared_W):
    n_tok, d = x.shape
    e_loc, _, h = expert_W.shape
    ch = n_tok // N_DEV

    def body(x_ref, rw_ref, idx_ref, ew_ref, sw_ref, out_ref,
             xg_ref, send_buf, rs_buf, ag_src, ag_buf,
             rs_send, rs_recv, ag_send, ag_recv):
        my = lax.axis_index("i")

        barrier_sem = pltpu.get_barrier_semaphore()
        for o in (1, 2, 3):
            pl.semaphore_signal(barrier_sem, inc=1,
                                device_id=((my + o) % N_DEV,),
                                device_id_type=pl.DeviceIdType.MESH)
        pl.semaphore_wait(barrier_sem, 3)

        xv = x_ref[:, :]

        scores = jnp.dot(xv, rw_ref[:, :], preferred_element_type=jnp.float32)
        m = jnp.max(scores, axis=-1, keepdims=True)
        p = jnp.exp(scores - m)
        p = p / jnp.sum(p, axis=-1, keepdims=True)
        idx = idx_ref[:, :]
        e_iota = lax.broadcasted_iota(jnp.int32, scores.shape, 1)
        gate = jnp.sum(jnp.where(e_iota == idx, p, 0.0), axis=-1, keepdims=True)
        xg_ref[:, :] = (xv * gate).astype(jnp.bfloat16)

        w = ew_ref[:, :, :].reshape(e_loc * d, h).astype(jnp.bfloat16)

        def chunk_partial(row0):
            xgc = xg_ref[pl.ds(row0, ch), :]
            idc = idx_ref[pl.ds(row0, ch), :]
            xsc = jnp.concatenate(
                [jnp.where(idc == (my * e_loc + le), xgc,
                           jnp.bfloat16(0.0)) for le in range(e_loc)],
                axis=1)
            return jnp.dot(xsc, w, preferred_element_type=jnp.float32)

        rs_rdmas = []
        for o in (1, 2, 3):
            peer = (my + o) % N_DEV
            send_buf[o - 1, :, :] = chunk_partial(peer * ch).astype(jnp.bfloat16)
            rdma = pltpu.make_async_remote_copy(
                src_ref=send_buf.at[o - 1],
                dst_ref=rs_buf.at[N_DEV - o],
                send_sem=rs_send.at[o - 1],
                recv_sem=rs_recv.at[N_DEV - o],
                device_id=(peer,),
                device_id_type=pl.DeviceIdType.MESH,
            )
            rdma.start()
            rs_rdmas.append(rdma)

        xseg = x_ref[pl.ds(my * ch, ch), :].astype(jnp.bfloat16)
        acc = chunk_partial(my * ch) + jnp.dot(
            xseg, sw_ref[:, :].astype(jnp.bfloat16),
            preferred_element_type=jnp.float32)

        for k in (1, 2, 3):
            recv = pltpu.make_async_remote_copy(
                src_ref=send_buf.at[0],
                dst_ref=rs_buf.at[k],
                send_sem=rs_send.at[0],
                recv_sem=rs_recv.at[k],
                device_id=(my,),
                device_id_type=pl.DeviceIdType.MESH,
            )
            recv.wait_recv()
            acc = acc + rs_buf[k, :, :].astype(jnp.float32)

        out_ref[pl.ds(my * ch, ch), :] = acc
        ag_src[:, :] = acc.astype(jnp.bfloat16)

        ag_rdmas = []
        for o in (1, 2, 3):
            peer = (my + o) % N_DEV
            rdma = pltpu.make_async_remote_copy(
                src_ref=ag_src,
                dst_ref=ag_buf.at[N_DEV - o],
                send_sem=ag_send.at[o - 1],
                recv_sem=ag_recv.at[N_DEV - o],
                device_id=(peer,),
                device_id_type=pl.DeviceIdType.MESH,
            )
            rdma.start()
            ag_rdmas.append(rdma)

        for k in (1, 2, 3):
            recv = pltpu.make_async_remote_copy(
                src_ref=ag_src,
                dst_ref=ag_buf.at[k],
                send_sem=ag_send.at[0],
                recv_sem=ag_recv.at[k],
                device_id=(my,),
                device_id_type=pl.DeviceIdType.MESH,
            )
            recv.wait_recv()
            out_ref[pl.ds(((my + k) % N_DEV) * ch, ch), :] = (
                ag_buf[k, :, :].astype(jnp.float32))

        for rdma in rs_rdmas + ag_rdmas:
            rdma.wait_send()

    return pl.pallas_call(
        body,
        out_shape=jax.ShapeDtypeStruct((n_tok, h), jnp.float32),
        in_specs=[pl.BlockSpec(memory_space=pltpu.VMEM)] * 5,
        out_specs=pl.BlockSpec(memory_space=pltpu.VMEM),
        scratch_shapes=[
            pltpu.VMEM((n_tok, d), jnp.bfloat16),
            pltpu.VMEM((3, ch, h), jnp.bfloat16),
            pltpu.VMEM((N_DEV, ch, h), jnp.bfloat16),
            pltpu.VMEM((ch, h), jnp.bfloat16),
            pltpu.VMEM((N_DEV, ch, h), jnp.bfloat16),
            pltpu.SemaphoreType.DMA((3,)),
            pltpu.SemaphoreType.DMA((N_DEV,)),
            pltpu.SemaphoreType.DMA((3,)),
            pltpu.SemaphoreType.DMA((N_DEV,)),
        ],
        compiler_params=pltpu.CompilerParams(collective_id=0),
    )(x, router_W, route_idx, expert_W, shared_W)


# device time: 25286 ns/iter; 3.4237x vs baseline; 1.0723x over previous
import jax
import jax.numpy as jnp
from jax import lax
from jax.experimental import pallas as pl
from jax.experimental.pallas import tpu as pltpu

N_DEV = 4
N_WAVE = 2
SEND_ORDER = (2, 1, 3)


def kernel(x, router_W, route_idx, expert_W, shared_W):
    n_tok, d = x.shape
    e_loc, _, h = expert_W.shape
    ch = n_tok // N_DEV
    hc = h // N_WAVE

    def body(x_ref, rw_ref, idx_ref, ew_ref, sw_ref, out_ref,
             xs_ref, send_buf, rs_buf, ag_src, ag_buf,
             rs_send, rs_recv, ag_send, ag_recv):
        my = lax.axis_index("i")

        barrier_sem = pltpu.get_barrier_semaphore()
        for o in (1, 2, 3):
            pl.semaphore_signal(barrier_sem, inc=1,
                                device_id=((my + o) % N_DEV,),
                                device_id_type=pl.DeviceIdType.MESH)
        pl.semaphore_wait(barrier_sem, 3)

        xv = x_ref[:, :]

        scores = jnp.dot(xv, rw_ref[:, :], preferred_element_type=jnp.float32)
        m = jnp.max(scores, axis=-1, keepdims=True)
        p = jnp.exp(scores - m)
        p = p / jnp.sum(p, axis=-1, keepdims=True)
        idx = idx_ref[:, :]
        e_iota = lax.broadcasted_iota(jnp.int32, scores.shape, 1)
        gate = jnp.sum(jnp.where(e_iota == idx, p, 0.0), axis=-1, keepdims=True)

        xg = (xv * gate).astype(jnp.bfloat16)
        xs_ref[:, :] = jnp.concatenate(
            [jnp.where(idx == (my * e_loc + le), xg, jnp.bfloat16(0.0))
             for le in range(e_loc)], axis=1)

        w = ew_ref[:, :, :].reshape(e_loc * d, h).astype(jnp.bfloat16)
        swb = sw_ref[:, :].astype(jnp.bfloat16)
        xseg = x_ref[pl.ds(my * ch, ch), :].astype(jnp.bfloat16)

        def chunk_partial(row0, v):
            xsc = xs_ref[pl.ds(row0, ch), :]
            return jnp.dot(xsc, w[:, v * hc:(v + 1) * hc],
                           preferred_element_type=jnp.float32)

        rs_rdmas = []
        for v in range(N_WAVE):
            for o in SEND_ORDER:
                peer = (my + o) % N_DEV
                send_buf[v, o - 1, :, :] = (
                    chunk_partial(peer * ch, v).astype(jnp.bfloat16))
                rdma = pltpu.make_async_remote_copy(
                    src_ref=send_buf.at[v, o - 1],
                    dst_ref=rs_buf.at[v, N_DEV - o],
                    send_sem=rs_send.at[v * 3 + o - 1],
                    recv_sem=rs_recv.at[v * N_DEV + N_DEV - o],
                    device_id=(peer,),
                    device_id_type=pl.DeviceIdType.MESH,
                )
                rdma.start()
                rs_rdmas.append(rdma)

        ag_rdmas = []
        for v in range(N_WAVE):
            acc = chunk_partial(my * ch, v) + jnp.dot(
                xseg, swb[:, v * hc:(v + 1) * hc],
                preferred_element_type=jnp.float32)
            for k in (1, 2, 3):
                recv = pltpu.make_async_remote_copy(
                    src_ref=send_buf.at[0, 0],
                    dst_ref=rs_buf.at[v, k],
                    send_sem=rs_send.at[0],
                    recv_sem=rs_recv.at[v * N_DEV + k],
                    device_id=(my,),
                    device_id_type=pl.DeviceIdType.MESH,
                )
                recv.wait_recv()
                acc = acc + rs_buf[v, k, :, :].astype(jnp.float32)

            out_ref[pl.ds(my * ch, ch), v * hc:(v + 1) * hc] = acc
            ag_src[v, :, :] = acc.astype(jnp.bfloat16)
            for o in SEND_ORDER:
                peer = (my + o) % N_DEV
                rdma = pltpu.make_async_remote_copy(
                    src_ref=ag_src.at[v],
                    dst_ref=ag_buf.at[v, N_DEV - o],
                    send_sem=ag_send.at[v * 3 + o - 1],
                    recv_sem=ag_recv.at[v * N_DEV + N_DEV - o],
                    device_id=(peer,),
                    device_id_type=pl.DeviceIdType.MESH,
                )
                rdma.start()
                ag_rdmas.append(rdma)

        for v in range(N_WAVE):
            for k in (1, 2, 3):
                recv = pltpu.make_async_remote_copy(
                    src_ref=ag_src.at[0],
                    dst_ref=ag_buf.at[v, k],
                    send_sem=ag_send.at[0],
                    recv_sem=ag_recv.at[v * N_DEV + k],
                    device_id=(my,),
                    device_id_type=pl.DeviceIdType.MESH,
                )
                recv.wait_recv()
                out_ref[pl.ds(((my + k) % N_DEV) * ch, ch),
                        v * hc:(v + 1) * hc] = (
                    ag_buf[v, k, :, :].astype(jnp.float32))

        for rdma in rs_rdmas + ag_rdmas:
            rdma.wait_send()

    return pl.pallas_call(
        body,
        out_shape=jax.ShapeDtypeStruct((n_tok, h), jnp.float32),
        in_specs=[pl.BlockSpec(memory_space=pltpu.VMEM)] * 5,
        out_specs=pl.BlockSpec(memory_space=pltpu.VMEM),
        scratch_shapes=[
            pltpu.VMEM((n_tok, e_loc * d), jnp.bfloat16),
            pltpu.VMEM((N_WAVE, 3, ch, hc), jnp.bfloat16),
            pltpu.VMEM((N_WAVE, N_DEV, ch, hc), jnp.bfloat16),
            pltpu.VMEM((N_WAVE, ch, hc), jnp.bfloat16),
            pltpu.VMEM((N_WAVE, N_DEV, ch, hc), jnp.bfloat16),
            pltpu.SemaphoreType.DMA((N_WAVE * 3,)),
            pltpu.SemaphoreType.DMA((N_WAVE * N_DEV,)),
            pltpu.SemaphoreType.DMA((N_WAVE * 3,)),
            pltpu.SemaphoreType.DMA((N_WAVE * N_DEV,)),
        ],
        compiler_params=pltpu.CompilerParams(collective_id=0),
    )(x, router_W, route_idx, expert_W, shared_W)


# device time: 24882 ns/iter; 3.4793x vs baseline; 1.0162x over previous
import jax
import jax.numpy as jnp
from jax import lax
from jax.experimental import pallas as pl
from jax.experimental.pallas import tpu as pltpu

N_DEV = 4
N_WAVE = 4
SEND_ORDER = (2, 1, 3)


def kernel(x, router_W, route_idx, expert_W, shared_W):
    n_tok, d = x.shape
    e_loc, _, h = expert_W.shape
    ch = n_tok // N_DEV
    hc = h // N_WAVE

    def body(x_ref, rw_ref, idx_ref, ew_ref, sw_ref, out_ref,
             xs_ref, send_buf, rs_buf, ag_src, ag_buf,
             rs_send, rs_recv, ag_send, ag_recv):
        my = lax.axis_index("i")

        barrier_sem = pltpu.get_barrier_semaphore()
        for o in (1, 2, 3):
            pl.semaphore_signal(barrier_sem, inc=1,
                                device_id=((my + o) % N_DEV,),
                                device_id_type=pl.DeviceIdType.MESH)

        xv = x_ref[:, :]

        scores = jnp.dot(xv, rw_ref[:, :], preferred_element_type=jnp.float32)
        m = jnp.max(scores, axis=-1, keepdims=True)
        p = jnp.exp(scores - m)
        p = p / jnp.sum(p, axis=-1, keepdims=True)
        idx = idx_ref[:, :]
        e_iota = lax.broadcasted_iota(jnp.int32, scores.shape, 1)
        gate = jnp.sum(jnp.where(e_iota == idx, p, 0.0), axis=-1, keepdims=True)

        xg = (xv * gate).astype(jnp.bfloat16)
        xs_ref[:, :] = jnp.concatenate(
            [jnp.where(idx == (my * e_loc + le), xg, jnp.bfloat16(0.0))
             for le in range(e_loc)], axis=1)

        w = ew_ref[:, :, :].reshape(e_loc * d, h).astype(jnp.bfloat16)
        swb = sw_ref[:, :].astype(jnp.bfloat16)
        xseg = x_ref[pl.ds(my * ch, ch), :].astype(jnp.bfloat16)

        def chunk_partial(row0, v):
            xsc = xs_ref[pl.ds(row0, ch), :]
            return jnp.dot(xsc, w[:, v * hc:(v + 1) * hc],
                           preferred_element_type=jnp.float32)

        pl.semaphore_wait(barrier_sem, 3)

        rs_rdmas = []
        for v in range(N_WAVE):
            for o in SEND_ORDER:
                peer = (my + o) % N_DEV
                send_buf[v, o - 1, :, :] = (
                    chunk_partial(peer * ch, v).astype(jnp.bfloat16))
                rdma = pltpu.make_async_remote_copy(
                    src_ref=send_buf.at[v, o - 1],
                    dst_ref=rs_buf.at[v, N_DEV - o],
                    send_sem=rs_send.at[v * 3 + o - 1],
                    recv_sem=rs_recv.at[v * N_DEV + N_DEV - o],
                    device_id=(peer,),
                    device_id_type=pl.DeviceIdType.MESH,
                )
                rdma.start()
                rs_rdmas.append(rdma)

        ag_rdmas = []
        for v in range(N_WAVE):
            acc = chunk_partial(my * ch, v) + jnp.dot(
                xseg, swb[:, v * hc:(v + 1) * hc],
                preferred_element_type=jnp.float32)
            for k in (1, 2, 3):
                recv = pltpu.make_async_remote_copy(
                    src_ref=send_buf.at[0, 0],
                    dst_ref=rs_buf.at[v, k],
                    send_sem=rs_send.at[0],
                    recv_sem=rs_recv.at[v * N_DEV + k],
                    device_id=(my,),
                    device_id_type=pl.DeviceIdType.MESH,
                )
                recv.wait_recv()
                acc = acc + rs_buf[v, k, :, :].astype(jnp.float32)

            out_ref[pl.ds(my * ch, ch), v * hc:(v + 1) * hc] = acc
            ag_src[v, :, :] = acc.astype(jnp.bfloat16)
            for o in SEND_ORDER:
                peer = (my + o) % N_DEV
                rdma = pltpu.make_async_remote_copy(
                    src_ref=ag_src.at[v],
                    dst_ref=ag_buf.at[v, N_DEV - o],
                    send_sem=ag_send.at[v * 3 + o - 1],
                    recv_sem=ag_recv.at[v * N_DEV + N_DEV - o],
                    device_id=(peer,),
                    device_id_type=pl.DeviceIdType.MESH,
                )
                rdma.start()
                ag_rdmas.append(rdma)

        for v in range(N_WAVE):
            for k in (1, 2, 3):
                recv = pltpu.make_async_remote_copy(
                    src_ref=ag_src.at[0],
                    dst_ref=ag_buf.at[v, k],
                    send_sem=ag_send.at[0],
                    recv_sem=ag_recv.at[v * N_DEV + k],
                    device_id=(my,),
                    device_id_type=pl.DeviceIdType.MESH,
                )
                recv.wait_recv()
                out_ref[pl.ds(((my + k) % N_DEV) * ch, ch),
                        v * hc:(v + 1) * hc] = (
                    ag_buf[v, k, :, :].astype(jnp.float32))

        for rdma in rs_rdmas + ag_rdmas:
            rdma.wait_send()

    return pl.pallas_call(
        body,
        out_shape=jax.ShapeDtypeStruct((n_tok, h), jnp.float32),
        in_specs=[pl.BlockSpec(memory_space=pltpu.VMEM)] * 5,
        out_specs=pl.BlockSpec(memory_space=pltpu.VMEM),
        scratch_shapes=[
            pltpu.VMEM((n_tok, e_loc * d), jnp.bfloat16),
            pltpu.VMEM((N_WAVE, 3, ch, hc), jnp.bfloat16),
            pltpu.VMEM((N_WAVE, N_DEV, ch, hc), jnp.bfloat16),
            pltpu.VMEM((N_WAVE, ch, hc), jnp.bfloat16),
            pltpu.VMEM((N_WAVE, N_DEV, ch, hc), jnp.bfloat16),
            pltpu.SemaphoreType.DMA((N_WAVE * 3,)),
            pltpu.SemaphoreType.DMA((N_WAVE * N_DEV,)),
            pltpu.SemaphoreType.DMA((N_WAVE * 3,)),
            pltpu.SemaphoreType.DMA((N_WAVE * N_DEV,)),
        ],
        compiler_params=pltpu.CompilerParams(collective_id=0),
    )(x, router_W, route_idx, expert_W, shared_W)


# device time: 20247 ns/iter; 4.2758x vs baseline; 1.2289x over previous
import jax
import jax.numpy as jnp
from jax import lax
from jax.experimental import pallas as pl
from jax.experimental.pallas import tpu as pltpu

N_DEV = 4
N_WAVE = 4
SEND_ORDER = (2, 1, 3)


def kernel(x, router_W, route_idx, expert_W, shared_W):
    n_tok, d = x.shape
    e_loc, _, h = expert_W.shape
    n_exp = e_loc * N_DEV
    ch = n_tok // N_DEV
    hc = h // N_WAVE

    xb = x.astype(jnp.bfloat16)
    scores = jnp.dot(xb, router_W.astype(jnp.bfloat16),
                     preferred_element_type=jnp.float32)
    m = jnp.max(scores, axis=-1, keepdims=True)
    p = jnp.exp(scores - m)
    p = p / jnp.sum(p, axis=-1, keepdims=True)
    e_iota = lax.broadcasted_iota(jnp.int32, (n_tok, n_exp), 1)
    gsel = jnp.where(e_iota == route_idx, p, 0.0).astype(jnp.bfloat16)
    wb = expert_W.astype(jnp.bfloat16).reshape(e_loc * d, h)
    swb = shared_W.astype(jnp.bfloat16)

    def body(xb_ref, gsel_ref, wb_ref, swb_ref, out_ref,
             send_buf, rs_buf, rs_send, rs_recv, ag_send, ag_recv):
        my = lax.axis_index("i")

        barrier_sem = pltpu.get_barrier_semaphore()
        for o in (1, 3):
            pl.semaphore_signal(barrier_sem, inc=1,
                                device_id=((my + o) % N_DEV,),
                                device_id_type=pl.DeviceIdType.MESH)

        lane_iota = lax.broadcasted_iota(jnp.int32, (ch, n_exp), 1)

        def chunk_lhs(row0):
            xbc = xb_ref[pl.ds(row0, ch), :]
            gc = gsel_ref[pl.ds(row0, ch), :]
            parts = []
            for le in range(e_loc):
                coef = jnp.sum(
                    jnp.where(lane_iota == (my * e_loc + le), gc,
                              jnp.bfloat16(0.0)),
                    axis=1, keepdims=True)
                parts.append(xbc * coef)
            return jnp.concatenate(parts, axis=1)

        w = wb_ref[:, :]
        pl.semaphore_wait(barrier_sem, 2)

        rs_rdmas = []
        for o in SEND_ORDER:
            peer = (my + o) % N_DEV
            xsc = chunk_lhs(peer * ch)
            for v in range(N_WAVE):
                send_buf[v, o - 1, :, :] = jnp.dot(
                    xsc, w[:, v * hc:(v + 1) * hc],
                    preferred_element_type=jnp.float32).astype(
                        jnp.float8_e4m3fn)
                rdma = pltpu.make_async_remote_copy(
                    src_ref=send_buf.at[v, o - 1],
                    dst_ref=rs_buf.at[v, N_DEV - o],
                    send_sem=rs_send.at[v * 3 + o - 1],
                    recv_sem=rs_recv.at[v * N_DEV + N_DEV - o],
                    device_id=(peer,),
                    device_id_type=pl.DeviceIdType.MESH,
                )
                rdma.start()
                rs_rdmas.append(rdma)

        xseg = xb_ref[pl.ds(my * ch, ch), :]
        own_lhs = chunk_lhs(my * ch)

        ag_rdmas = []
        for v in range(N_WAVE):
            acc = jnp.dot(own_lhs, w[:, v * hc:(v + 1) * hc],
                          preferred_element_type=jnp.float32) + jnp.dot(
                xseg, swb_ref[:, v * hc:(v + 1) * hc],
                preferred_element_type=jnp.float32)
            for k in (1, 3, 2):
                recv = pltpu.make_async_remote_copy(
                    src_ref=send_buf.at[0, 0],
                    dst_ref=rs_buf.at[v, k],
                    send_sem=rs_send.at[0],
                    recv_sem=rs_recv.at[v * N_DEV + k],
                    device_id=(my,),
                    device_id_type=pl.DeviceIdType.MESH,
                )
                recv.wait_recv()
                acc = acc + rs_buf[v, k, :, :].astype(jnp.float32)

            out_ref[pl.ds(my * ch, ch), pl.ds(v * hc, hc)] = (
                acc.astype(jnp.bfloat16))
            for o in SEND_ORDER:
                peer = (my + o) % N_DEV
                rdma = pltpu.make_async_remote_copy(
                    src_ref=out_ref.at[pl.ds(my * ch, ch), pl.ds(v * hc, hc)],
                    dst_ref=out_ref.at[pl.ds(my * ch, ch), pl.ds(v * hc, hc)],
                    send_sem=ag_send.at[v * 3 + o - 1],
                    recv_sem=ag_recv.at[v * N_DEV + N_DEV - o],
                    device_id=(peer,),
                    device_id_type=pl.DeviceIdType.MESH,
                )
                rdma.start()
                ag_rdmas.append(rdma)

        for v in range(N_WAVE):
            for k in (1, 3, 2):
                recv = pltpu.make_async_remote_copy(
                    src_ref=out_ref.at[pl.ds(0, ch), pl.ds(v * hc, hc)],
                    dst_ref=out_ref.at[pl.ds(((my + k) % N_DEV) * ch, ch),
                                       pl.ds(v * hc, hc)],
                    send_sem=ag_send.at[0],
                    recv_sem=ag_recv.at[v * N_DEV + k],
                    device_id=(my,),
                    device_id_type=pl.DeviceIdType.MESH,
                )
                recv.wait_recv()

        for rdma in rs_rdmas + ag_rdmas:
            rdma.wait_send()

    return pl.pallas_call(
        body,
        out_shape=jax.ShapeDtypeStruct((n_tok, h), jnp.bfloat16),
        in_specs=[pl.BlockSpec(memory_space=pltpu.VMEM)] * 4,
        out_specs=pl.BlockSpec(memory_space=pltpu.VMEM),
        scratch_shapes=[
            pltpu.VMEM((N_WAVE, 3, ch, hc), jnp.float8_e4m3fn),
            pltpu.VMEM((N_WAVE, N_DEV, ch, hc), jnp.float8_e4m3fn),
            pltpu.SemaphoreType.DMA((N_WAVE * 3,)),
            pltpu.SemaphoreType.DMA((N_WAVE * N_DEV,)),
            pltpu.SemaphoreType.DMA((N_WAVE * 3,)),
            pltpu.SemaphoreType.DMA((N_WAVE * N_DEV,)),
        ],
        compiler_params=pltpu.CompilerParams(collective_id=0),
    )(xb, gsel, wb, swb)
